# baseline (speedup 1.0000x reference)
import jax
import jax.numpy as jnp
from jax import lax
from jax.experimental import pallas as pl
from jax.experimental.pallas import tpu as pltpu
from jax.experimental.pallas import tpu_sc as plsc

_NC = 2
_NS = 16
_NW = _NC * _NS
_LANE = 16


def _rsqrt_vec(x):
    x = jnp.maximum(x, jnp.float32(1e-24))
    i = lax.bitcast_convert_type(x, jnp.int32)
    i = jnp.int32(0x5F3759DF) - (i >> 1)
    y = lax.bitcast_convert_type(i, jnp.float32)
    for _ in range(3):
        y = y * (jnp.float32(1.5) - jnp.float32(0.5) * x * y * y)
    return y


def _tree_sum(vs):
    while len(vs) > 1:
        vs = [a + b for a, b in zip(vs[::2], vs[1::2])] + (
            [vs[-1]] if len(vs) % 2 else [])
    return vs[0]


def _sumsq_and_dot(ref, row, es):
    acc_d = None
    acc_w = None
    for k in range(len(es)):
        w = ref[row, pl.ds(k * _LANE, _LANE)]
        d = w * es[k]
        q = w * w
        acc_d = d if acc_d is None else acc_d + d
        acc_w = q if acc_w is None else acc_w + q
    return jnp.sum(acc_d), jnp.sum(acc_w)


def _make_sc_kernel(B, L, LP, D):
    NB = B // _NW
    CR = L
    NCHUNK = NB
    RING = 8
    OUT_W = NB * LP
    KD = D // _LANE
    NGF = L // _LANE
    NREM = L - NGF * _LANE

    def body(wids_hbm, eids_hbm, wtab_hbm, etab_hbm, out_hbm,
             widx_v, eidx_v, erows_v,
             wb0, wb1, wb2, wb3, wb4, wb5, wb6, wb7, dots_v,
             sem, s0, s1, s2, s3, s4, s5, s6, s7):
        wbuf = [wb0, wb1, wb2, wb3, wb4, wb5, wb6, wb7]
        wsem = [s0, s1, s2, s3, s4, s5, s6, s7]
        wid = lax.axis_index("s") * _NC + lax.axis_index("c")
        lanes = lax.iota(jnp.int32, _LANE)

        def start(c, wref, s):
            pltpu.async_copy(wtab_hbm.at[widx_v.at[c]], wref, s)

        def wait(c, wref, s):
            pltpu.make_async_copy(wtab_hbm.at[widx_v.at[c]], wref, s).wait()

        pltpu.sync_copy(eids_hbm.at[pl.ds(wid * NB, NB)], eidx_v)
        pltpu.sync_copy(wids_hbm.at[pl.ds(wid * NCHUNK, NCHUNK)], widx_v)
        for u in range(RING - 1):
            start(u, wbuf[u], wsem[u])
        pltpu.async_copy(etab_hbm.at[eidx_v], erows_v, sem).wait()

        def ent_grp(g, _):
            vn = jnp.zeros((_LANE,), jnp.float32)
            for j in range(_LANE):
                b = g * _LANE + j
                evs = [erows_v[b, pl.ds(k * _LANE, _LANE)]
                       for k in range(KD)]
                acc = _tree_sum([e * e for e in evs])
                s = jnp.full((_LANE,), jnp.sum(acc), dtype=jnp.float32)
                vn = jnp.where(lanes == j, s, vn)
            rinv = _rsqrt_vec(vn)
            for j in range(_LANE):
                b = g * _LANE + j
                sv = jnp.full((_LANE,), rinv[j], dtype=jnp.float32)
                for k in range(KD):
                    sl = pl.ds(k * _LANE, _LANE)
                    erows_v[b, sl] = erows_v[b, sl] * sv
            return 0

        lax.fori_loop(0, NB // _LANE, ent_grp, 0)

        def compute_chunk(c, wref):
            es = [erows_v[c, pl.ds(k * _LANE, _LANE)] for k in range(KD)]
            for g in range(NGF + 1):
                nrows = _LANE if g < NGF else NREM
                vd = jnp.zeros((_LANE,), jnp.float32)
                vw = jnp.ones((_LANE,), jnp.float32)
                for j in range(nrows):
                    sd, sw = _sumsq_and_dot(wref, g * _LANE + j, es)
                    m = lanes == j
                    vd = jnp.where(
                        m, jnp.full((_LANE,), sd, dtype=jnp.float32), vd)
                    vw = jnp.where(
                        m, jnp.full((_LANE,), sw, dtype=jnp.float32), vw)
                dots_v[pl.ds(c * LP + g * _LANE, _LANE)] = (
                    vd * _rsqrt_vec(vw))

        def ring(i, _):
            for u in range(RING):
                c = i * RING + u

                @pl.when(c + RING - 1 < NCHUNK)
                def _(c=c, u=u):
                    start(c + RING - 1, wbuf[(u + RING - 1) % RING],
                          wsem[(u + RING - 1) % RING])

                wait(c, wbuf[u], wsem[u])
                compute_chunk(c, wbuf[u])
            return 0

        lax.fori_loop(0, NCHUNK // RING, ring, 0)

        pltpu.sync_copy(dots_v, out_hbm.at[pl.ds(wid * OUT_W, OUT_W)])

    mesh = plsc.VectorSubcoreMesh(core_axis_name="c", subcore_axis_name="s")
    return pl.kernel(
        body,
        mesh=mesh,
        compiler_params=pltpu.CompilerParams(needs_layout_passes=False),
        out_type=jax.ShapeDtypeStruct((B * LP,), jnp.float32),
        scratch_types=[
            pltpu.VMEM((NCHUNK, CR), jnp.int32),
            pltpu.VMEM((NB,), jnp.int32),
            pltpu.VMEM((NB, D), jnp.float32),
        ] + [pltpu.VMEM((CR, D), jnp.float32)] * 8 + [
            pltpu.VMEM((OUT_W,), jnp.float32),
        ] + [pltpu.SemaphoreType.DMA] * 9,
    )


def kernel(batch_size, word_ids, entity_ids, word_table, entity_table):
    B, L = word_ids.shape
    D = word_table.shape[1]
    LP = -(-L // _LANE) * _LANE
    wids = word_ids.astype(jnp.int32)
    eids = entity_ids.astype(jnp.int32)
    f = _make_sc_kernel(B, L, LP, D)
    out = f(wids, eids, word_table, entity_table)
    return out.reshape(B, LP)[:, :L, None]


# speedup vs baseline: 1.1938x; 1.1938x over previous
import jax
import jax.numpy as jnp
from jax import lax
from jax.experimental import pallas as pl
from jax.experimental.pallas import tpu as pltpu
from jax.experimental.pallas import tpu_sc as plsc

_NC = 2
_NS = 16
_NW = _NC * _NS
_LANE = 16


def _rsqrt_vec(x):
    x = jnp.maximum(x, jnp.float32(1e-24))
    i = lax.bitcast_convert_type(x, jnp.int32)
    i = jnp.int32(0x5F3759DF) - (i >> 1)
    y = lax.bitcast_convert_type(i, jnp.float32)
    for _ in range(3):
        y = y * (jnp.float32(1.5) - jnp.float32(0.5) * x * y * y)
    return y


def _tree_sum(vs):
    while len(vs) > 1:
        vs = [a + b for a, b in zip(vs[::2], vs[1::2])] + (
            [vs[-1]] if len(vs) % 2 else [])
    return vs[0]


def _sumsq_and_dot(ref, row, es):
    acc_d = None
    acc_w = None
    for k in range(len(es)):
        w = ref[row, pl.ds(k * _LANE, _LANE)]
        d = w * es[k]
        q = w * w
        acc_d = d if acc_d is None else acc_d + d
        acc_w = q if acc_w is None else acc_w + q
    return jnp.sum(acc_d), jnp.sum(acc_w)


def _make_sc_kernel(B, L, LP, D):
    NB = B // _NW
    CR = L
    NCHUNK = NB
    RING = 4
    OUT_W = NB * LP
    KD = D // _LANE
    NGF = L // _LANE
    NREM = L - NGF * _LANE

    def body(wids_hbm, eids_hbm, wtab_hbm, etab_hbm, out_hbm,
             widx_v, eidx_v, erows_v,
             wb0, wb1, wb2, wb3, dots_v,
             sem, s0, s1, s2, s3):
        wbuf = [wb0, wb1, wb2, wb3]
        wsem = [s0, s1, s2, s3]
        wid = lax.axis_index("s") * _NC + lax.axis_index("c")
        lanes = lax.iota(jnp.int32, _LANE)

        def start(c, wref, s):
            pltpu.async_copy(wtab_hbm.at[widx_v.at[c]], wref, s)

        def wait(c, wref, s):
            pltpu.make_async_copy(wtab_hbm.at[widx_v.at[c]], wref, s).wait()

        pltpu.sync_copy(eids_hbm.at[pl.ds(wid * NB, NB)], eidx_v)
        pltpu.sync_copy(wids_hbm.at[pl.ds(wid * NCHUNK, NCHUNK)], widx_v)
        for u in range(RING - 1):
            start(u, wbuf[u], wsem[u])
        pltpu.async_copy(etab_hbm.at[eidx_v], erows_v, sem).wait()

        def ent_grp(g, _):
            vn = jnp.zeros((_LANE,), jnp.float32)
            for j in range(_LANE):
                b = g * _LANE + j
                evs = [erows_v[b, pl.ds(k * _LANE, _LANE)]
                       for k in range(KD)]
                acc = _tree_sum([e * e for e in evs])
                s = jnp.full((_LANE,), jnp.sum(acc), dtype=jnp.float32)
                vn = jnp.where(lanes == j, s, vn)
            rinv = _rsqrt_vec(vn)
            for j in range(_LANE):
                b = g * _LANE + j
                sv = jnp.full((_LANE,), rinv[j], dtype=jnp.float32)
                for k in range(KD):
                    sl = pl.ds(k * _LANE, _LANE)
                    erows_v[b, sl] = erows_v[b, sl] * sv
            return 0

        lax.fori_loop(0, NB // _LANE, ent_grp, 0)

        def compute_chunk(c, wref):
            es = [erows_v[c, pl.ds(k * _LANE, _LANE)] for k in range(KD)]
            for g in range(NGF + 1):
                nrows = _LANE if g < NGF else NREM
                vd = jnp.zeros((_LANE,), jnp.float32)
                vw = jnp.ones((_LANE,), jnp.float32)
                for j in range(nrows):
                    sd, sw = _sumsq_and_dot(wref, g * _LANE + j, es)
                    m = lanes == j
                    vd = jnp.where(
                        m, jnp.full((_LANE,), sd, dtype=jnp.float32), vd)
                    vw = jnp.where(
                        m, jnp.full((_LANE,), sw, dtype=jnp.float32), vw)
                dots_v[pl.ds(c * LP + g * _LANE, _LANE)] = (
                    vd * _rsqrt_vec(vw))

        def ring(i, _):
            for u in range(RING):
                c = i * RING + u

                @pl.when(c + RING - 1 < NCHUNK)
                def _(c=c, u=u):
                    start(c + RING - 1, wbuf[(u + RING - 1) % RING],
                          wsem[(u + RING - 1) % RING])

                wait(c, wbuf[u], wsem[u])
                compute_chunk(c, wbuf[u])
            return 0

        lax.fori_loop(0, NCHUNK // RING, ring, 0)

        pltpu.sync_copy(dots_v, out_hbm.at[pl.ds(wid * OUT_W, OUT_W)])

    mesh = plsc.VectorSubcoreMesh(core_axis_name="c", subcore_axis_name="s")
    return pl.kernel(
        body,
        mesh=mesh,
        compiler_params=pltpu.CompilerParams(needs_layout_passes=False),
        out_type=jax.ShapeDtypeStruct((B * LP,), jnp.float32),
        scratch_types=[
            pltpu.VMEM((NCHUNK, CR), jnp.int32),
            pltpu.VMEM((NB,), jnp.int32),
            pltpu.VMEM((NB, D), jnp.float32),
        ] + [pltpu.VMEM((CR, D), jnp.float32)] * 4 + [
            pltpu.VMEM((OUT_W,), jnp.float32),
        ] + [pltpu.SemaphoreType.DMA] * 5,
    )


def kernel(batch_size, word_ids, entity_ids, word_table, entity_table):
    B, L = word_ids.shape
    D = word_table.shape[1]
    LP = -(-L // _LANE) * _LANE
    wids = word_ids.astype(jnp.int32)
    eids = entity_ids.astype(jnp.int32)
    f = _make_sc_kernel(B, L, LP, D)
    out = f(wids, eids, word_table, entity_table)
    return out.reshape(B, LP)[:, :L, None]
